# baseline (speedup 1.0000x reference)
import jax
import jax.numpy as jnp
from jax import lax
from jax.experimental import pallas as pl
from jax.experimental.pallas import tpu as pltpu
from jax.experimental.pallas import tpu_sc as plsc

_N = 10000
_DIN = 256
_DH = 512
_NC = 2
_NS = 16
_B = 64
_NSLOT = 4
_NACC = 10112
_RPT = _NACC // _NS
_SG = 4
_MBLK = 1000
def _sc_segment_sum(n_blocks, n_chunks, with_cnt):
  feat_rounds = n_blocks // _NC
  rounds = feat_rounds + (1 if with_cnt else 0)
  n_sg = n_chunks // _SG

  def body(h_ref, ei_ref, zf_ref, ones_ref, agg_out,
           ibuf, msg, *sems_and_agg):
    gsem = sems_and_agg[:_NSLOT]
    ssem = sems_and_agg[_NSLOT:2 * _NSLOT]
    agg_sh = sems_and_agg[-1]
    cid = lax.axis_index("c")
    sid = lax.axis_index("s")
    r0 = sid * _RPT

    def run_gather_block(f):
      ebase = (f * _NS + sid) * n_chunks
      span = 2 * _SG

      def it(t, carry):
        for q in range(2):
          pltpu.sync_copy(
              ei_ref.at[pl.ds(ebase + (2 * t + q) * _SG, _SG)], ibuf.at[q])
        gd = {}
        sd = {}
        for k in range(_NSLOT):
          q, j = divmod(k, _SG)
          gd[k] = pltpu.async_copy(
              h_ref.at[ibuf.at[q, j, 0]], msg.at[k], gsem[k])
        for k in range(span):
          q, j = divmod(k, _SG)
          s = k % _NSLOT
          gd[k].wait()
          sd[k] = pltpu.async_copy(
              msg.at[s], agg_sh.at[ibuf.at[q, j, 1]], ssem[s], add=True)
          if k + _NSLOT < span:
            sd[k].wait()
            nq, nj = divmod(k + _NSLOT, _SG)
            gd[k + _NSLOT] = pltpu.async_copy(
                h_ref.at[ibuf.at[nq, nj, 0]], msg.at[s], gsem[s])
        for k in range(span - _NSLOT, span):
          sd[k].wait()
        return carry
      lax.fori_loop(0, n_sg // 2, it, 0)

    def run_cnt(c_lo, c_hi):
      ebase = sid * n_chunks
      pltpu.sync_copy(ones_ref, msg.at[0])

      def it(t, carry):
        pltpu.sync_copy(
            ei_ref.at[pl.ds(ebase + c_lo + t * _SG, _SG)], ibuf.at[0])
        descs = [pltpu.async_copy(msg.at[0], agg_sh.at[ibuf.at[0, j, 1]],
                                  ssem[0], add=True) for j in range(_SG)]
        for d in descs:
          d.wait()
        return carry
      lax.fori_loop(0, (c_hi - c_lo) // _SG, it, 0)

    for r in range(rounds):
      pltpu.sync_copy(zf_ref, agg_sh.at[pl.ds(r0, _RPT)])
      plsc.subcore_barrier()
      is_cnt = with_cnt and r == feat_rounds
      if is_cnt:
        half = n_chunks // 2
        pl.when(cid == 0)(lambda: run_cnt(0, half))
        pl.when(cid == 1)(lambda: run_cnt(half, n_chunks))
      else:
        pl.when(cid == 0)(lambda: run_gather_block(2 * r))
        pl.when(cid == 1)(lambda: run_gather_block(2 * r + 1))
      plsc.subcore_barrier()
      for c in range(_NC):
        f = 2 * r + c
        pl.when(cid == c)(lambda f=f: pltpu.sync_copy(
            agg_sh.at[pl.ds(r0, _RPT)], agg_out.at[f].at[pl.ds(r0, _RPT)]))

  out_type = jax.ShapeDtypeStruct((2 * rounds, _NACC, 128), jnp.float32)
  scratch = (
      [pltpu.VMEM((2, _SG, 2, _B), jnp.int32),
       pltpu.VMEM((_NSLOT, _B, 128), jnp.float32)]
      + [pltpu.SemaphoreType.DMA] * (2 * _NSLOT)
      + [pltpu.VMEM_SHARED((_NACC, 128), jnp.float32)]
  )
  mesh = plsc.VectorSubcoreMesh(core_axis_name="c", subcore_axis_name="s")
  return pl.kernel(body, out_type=out_type, mesh=mesh,
                   scratch_types=scratch, name=f"sc_segsum_{n_blocks}")


def _tc_layer1(a1, cnt, x, w1n, w1r, b1):
  m = _MBLK

  def body(a_ref, c_ref, x_ref, wn_ref, wr_ref, b_ref, out_ref):
    cnt = c_ref[0][:, :1] + c_ref[1][:, :1]
    inv = 1.0 / jnp.maximum(cnt, 1.0)
    agg = jnp.concatenate([a_ref[0], a_ref[1]], axis=1) * inv
    z = (jnp.dot(agg, wn_ref[...], preferred_element_type=jnp.float32)
         + jnp.dot(x_ref[...], wr_ref[...], preferred_element_type=jnp.float32)
         + b_ref[...])
    h = jnp.maximum(z, 0.0)
    for j in range(4):
      out_ref[j] = h[:, j * 128:(j + 1) * 128]

  return pl.pallas_call(
      body,
      grid=(_N // m,),
      in_specs=[
          pl.BlockSpec((2, m, 128), lambda i: (0, i, 0)),
          pl.BlockSpec((2, m, 128), lambda i: (1, i, 0)),
          pl.BlockSpec((m, _DIN), lambda i: (i, 0)),
          pl.BlockSpec((_DIN, _DH), lambda i: (0, 0)),
          pl.BlockSpec((_DIN, _DH), lambda i: (0, 0)),
          pl.BlockSpec((1, _DH), lambda i: (0, 0)),
      ],
      out_specs=pl.BlockSpec((4, m, 128), lambda i: (0, i, 0)),
      out_shape=jax.ShapeDtypeStruct((4, _N, 128), jnp.float32),
      name="tc_layer1",
  )(a1, cnt, x, w1n, w1r, b1)


def _tc_layer2(a2, cnt, h1b, w2n, w2r, b2, wc, bc):
  m = _MBLK

  def body(a_ref, c_ref, h_ref, wn_ref, wr_ref, b_ref, wc_ref, bc_ref,
           out_ref):
    cnt = c_ref[0][:, :1] + c_ref[1][:, :1]
    inv = 1.0 / jnp.maximum(cnt, 1.0)
    agg = jnp.concatenate([a_ref[j] for j in range(4)], axis=1) * inv
    h1 = jnp.concatenate([h_ref[j] for j in range(4)], axis=1)
    z = (jnp.dot(agg, wn_ref[...], preferred_element_type=jnp.float32)
         + jnp.dot(h1, wr_ref[...], preferred_element_type=jnp.float32)
         + b_ref[...])
    h2 = jnp.maximum(z, 0.0)
    out_ref[...] = (jnp.dot(h2, wc_ref[...], preferred_element_type=jnp.float32)
                    + bc_ref[0, 0])

  return pl.pallas_call(
      body,
      grid=(_N // m,),
      in_specs=[
          pl.BlockSpec((4, m, 128), lambda i: (0, i, 0)),
          pl.BlockSpec((2, m, 128), lambda i: (1, i, 0)),
          pl.BlockSpec((4, m, 128), lambda i: (0, i, 0)),
          pl.BlockSpec((_DH, _DH), lambda i: (0, 0)),
          pl.BlockSpec((_DH, _DH), lambda i: (0, 0)),
          pl.BlockSpec((1, _DH), lambda i: (0, 0)),
          pl.BlockSpec((_DH, 128), lambda i: (0, 0)),
          pl.BlockSpec((1, 1), lambda i: (0, 0)),
      ],
      out_specs=pl.BlockSpec((m, 128), lambda i: (i, 0)),
      out_shape=jax.ShapeDtypeStruct((_N, 128), jnp.float32),
      name="tc_layer2",
  )(a2, cnt, h1b, w2n, w2r, b2, wc, bc)


def kernel(x, edge_index, W1_neigh, W1_root, b1, W2_neigh, W2_root, b2, Wc, bc):
  src = edge_index[0]
  dst = edge_index[1]
  e = src.shape[0]
  n_chunks = -(-e // (_NS * _B * 2 * _SG)) * 2 * _SG
  pad = n_chunks * _NS * _B - e
  if pad:
    src = jnp.concatenate([src, jnp.zeros((pad,), jnp.int32)])
    dst = jnp.concatenate([dst, jnp.full((pad,), _N, jnp.int32)])
  src3 = src.reshape(_NS, n_chunks, _B)
  dst3 = dst.reshape(_NS, n_chunks, _B)

  def edge_rows(f_blocks):
    off = (jnp.arange(f_blocks, dtype=jnp.int32) * _N)[:, None, None, None]
    sa = src3[None] + off
    da = jnp.broadcast_to(dst3[None], sa.shape)
    return jnp.stack([sa, da], axis=3).reshape(-1, 2, _B)

  ei1 = edge_rows(2)
  ei2 = edge_rows(4)
  zf = jnp.zeros((_RPT, 128), jnp.float32)
  ones = jnp.ones((_B, 128), jnp.float32)
  xb = x.reshape(_N, 2, 128).transpose(1, 0, 2).reshape(2 * _N, 128)

  out1 = _sc_segment_sum(2, n_chunks, True)(xb, ei1, zf, ones)
  h1b = _tc_layer1(out1, out1, x, W1_neigh, W1_root, b1.reshape(1, _DH))
  agg2 = _sc_segment_sum(4, n_chunks, False)(
      h1b.reshape(4 * _N, 128), ei2, zf, ones)
  wc_pad = jnp.pad(Wc, ((0, 0), (0, 127)))
  out = _tc_layer2(agg2, out1, h1b, W2_neigh, W2_root, b2.reshape(1, _DH),
                   wc_pad, bc.reshape(1, 1))
  return out[:, 0]


# speedup vs baseline: 1.0723x; 1.0723x over previous
import jax
import jax.numpy as jnp
from jax import lax
from jax.experimental import pallas as pl
from jax.experimental.pallas import tpu as pltpu
from jax.experimental.pallas import tpu_sc as plsc

_N = 10000
_DIN = 256
_DH = 512
_NC = 2
_NS = 16
_B = 128
_NSLOT = 2
_SG = 4
_NACC = 10112
_RPT = _NACC // _NS
_MBLK = 1000


def _sc_segment_sum(n_blocks, n_chunks, with_cnt):
  feat_rounds = n_blocks // _NC
  rounds = feat_rounds + (1 if with_cnt else 0)
  n_sg = n_chunks // _SG
  span = 2 * _SG

  def body(h_ref, ei_ref, zf_ref, ones_ref, agg_out,
           ibuf, msg, *sems_and_agg):
    gsem = sems_and_agg[:_NSLOT]
    ssem = sems_and_agg[_NSLOT:2 * _NSLOT]
    agg_sh = sems_and_agg[-1]
    cid = lax.axis_index("c")
    sid = lax.axis_index("s")
    r0 = sid * _RPT

    def run_gather_block(f):
      ebase = (f * _NS + sid) * n_chunks

      def it(t, carry):
        for q in range(2):
          pltpu.sync_copy(
              ei_ref.at[pl.ds(ebase + (2 * t + q) * _SG, _SG)], ibuf.at[q])
        gd = {}
        sd = {}
        for k in range(_NSLOT):
          q, j = divmod(k, _SG)
          gd[k] = pltpu.async_copy(
              h_ref.at[ibuf.at[q, j, 0]], msg.at[k], gsem[k])
        for k in range(span):
          q, j = divmod(k, _SG)
          s = k % _NSLOT
          gd[k].wait()
          sd[k] = pltpu.async_copy(
              msg.at[s], agg_sh.at[ibuf.at[q, j, 1]], ssem[s], add=True)
          if k + _NSLOT < span:
            sd[k].wait()
            nq, nj = divmod(k + _NSLOT, _SG)
            gd[k + _NSLOT] = pltpu.async_copy(
                h_ref.at[ibuf.at[nq, nj, 0]], msg.at[s], gsem[s])
        for k in range(span - _NSLOT, span):
          sd[k].wait()
        return carry
      lax.fori_loop(0, n_sg // 2, it, 0)

    def run_cnt(c_lo, c_hi):
      ebase = sid * n_chunks
      pltpu.sync_copy(ones_ref, msg.at[0])

      def it(t, carry):
        pltpu.sync_copy(
            ei_ref.at[pl.ds(ebase + c_lo + t * _SG, _SG)], ibuf.at[0])
        descs = [pltpu.async_copy(msg.at[0], agg_sh.at[ibuf.at[0, j, 1]],
                                  ssem[0], add=True) for j in range(_SG)]
        for d in descs:
          d.wait()
        return carry
      lax.fori_loop(0, (c_hi - c_lo) // _SG, it, 0)

    for r in range(rounds):
      pltpu.sync_copy(zf_ref, agg_sh.at[pl.ds(r0, _RPT)])
      plsc.subcore_barrier()
      is_cnt = with_cnt and r == feat_rounds
      if is_cnt:
        half = n_chunks // 2
        pl.when(cid == 0)(lambda: run_cnt(0, half))
        pl.when(cid == 1)(lambda: run_cnt(half, n_chunks))
      else:
        pl.when(cid == 0)(lambda: run_gather_block(2 * r))
        pl.when(cid == 1)(lambda: run_gather_block(2 * r + 1))
      plsc.subcore_barrier()
      for c in range(_NC):
        f = 2 * r + c
        pl.when(cid == c)(lambda f=f: pltpu.sync_copy(
            agg_sh.at[pl.ds(r0, _RPT)], agg_out.at[f].at[pl.ds(r0, _RPT)]))

  out_type = jax.ShapeDtypeStruct((2 * rounds, _NACC, 128), jnp.float32)
  scratch = (
      [pltpu.VMEM((2, _SG, 2, _B), jnp.int32),
       pltpu.VMEM((_NSLOT, _B, 128), jnp.float32)]
      + [pltpu.SemaphoreType.DMA] * (2 * _NSLOT)
      + [pltpu.VMEM_SHARED((_NACC, 128), jnp.float32)]
  )
  mesh = plsc.VectorSubcoreMesh(core_axis_name="c", subcore_axis_name="s")
  return pl.kernel(body, out_type=out_type, mesh=mesh,
                   scratch_types=scratch, name=f"sc_segsum_{n_blocks}")


def _tc_root(h, w, b, din):
  m = _MBLK

  def body(h_ref, w_ref, b_ref, out_ref):
    out_ref[...] = (jnp.dot(h_ref[...], w_ref[...],
                            preferred_element_type=jnp.float32) + b_ref[...])

  return pl.pallas_call(
      body,
      grid=(_N // m,),
      in_specs=[
          pl.BlockSpec((m, din), lambda i: (i, 0)),
          pl.BlockSpec((din, _DH), lambda i: (0, 0)),
          pl.BlockSpec((1, _DH), lambda i: (0, 0)),
      ],
      out_specs=pl.BlockSpec((m, _DH), lambda i: (i, 0)),
      out_shape=jax.ShapeDtypeStruct((_N, _DH), jnp.float32),
      name="tc_root",
  )(h, w, b)


def _tc_root2(h1b, w, b):
  m = _MBLK

  def body(h_ref, w_ref, b_ref, out_ref):
    h1 = jnp.concatenate([h_ref[j] for j in range(4)], axis=1)
    out_ref[...] = (jnp.dot(h1, w_ref[...],
                            preferred_element_type=jnp.float32) + b_ref[...])

  return pl.pallas_call(
      body,
      grid=(_N // m,),
      in_specs=[
          pl.BlockSpec((4, m, 128), lambda i: (0, i, 0)),
          pl.BlockSpec((_DH, _DH), lambda i: (0, 0)),
          pl.BlockSpec((1, _DH), lambda i: (0, 0)),
      ],
      out_specs=pl.BlockSpec((m, _DH), lambda i: (i, 0)),
      out_shape=jax.ShapeDtypeStruct((_N, _DH), jnp.float32),
      name="tc_root2",
  )(h1b, w, b)


def _tc_main1(a1, cnt, zr, w1n):
  m = _MBLK

  def body(a_ref, c_ref, z_ref, wn_ref, out_ref):
    cnt = c_ref[0][:, :1] + c_ref[1][:, :1]
    inv = 1.0 / jnp.maximum(cnt, 1.0)
    agg = jnp.concatenate([a_ref[0], a_ref[1]], axis=1) * inv
    z = (jnp.dot(agg, wn_ref[...], preferred_element_type=jnp.float32)
         + z_ref[...])
    h = jnp.maximum(z, 0.0)
    for j in range(4):
      out_ref[j] = h[:, j * 128:(j + 1) * 128]

  return pl.pallas_call(
      body,
      grid=(_N // m,),
      in_specs=[
          pl.BlockSpec((2, m, 128), lambda i: (0, i, 0)),
          pl.BlockSpec((2, m, 128), lambda i: (1, i, 0)),
          pl.BlockSpec((m, _DH), lambda i: (i, 0)),
          pl.BlockSpec((_DIN, _DH), lambda i: (0, 0)),
      ],
      out_specs=pl.BlockSpec((4, m, 128), lambda i: (0, i, 0)),
      out_shape=jax.ShapeDtypeStruct((4, _N, 128), jnp.float32),
      name="tc_main1",
  )(a1, cnt, zr, w1n)


def _tc_main2(a2, cnt, zr, w2n, wc, bc):
  m = _MBLK

  def body(a_ref, c_ref, z_ref, wn_ref, wc_ref, bc_ref, out_ref):
    cnt = c_ref[0][:, :1] + c_ref[1][:, :1]
    inv = 1.0 / jnp.maximum(cnt, 1.0)
    agg = jnp.concatenate([a_ref[j] for j in range(4)], axis=1) * inv
    z = (jnp.dot(agg, wn_ref[...], preferred_element_type=jnp.float32)
         + z_ref[...])
    h2 = jnp.maximum(z, 0.0)
    out_ref[...] = (jnp.dot(h2, wc_ref[...], preferred_element_type=jnp.float32)
                    + bc_ref[0, 0])

  return pl.pallas_call(
      body,
      grid=(_N // m,),
      in_specs=[
          pl.BlockSpec((4, m, 128), lambda i: (0, i, 0)),
          pl.BlockSpec((2, m, 128), lambda i: (1, i, 0)),
          pl.BlockSpec((m, _DH), lambda i: (i, 0)),
          pl.BlockSpec((_DH, _DH), lambda i: (0, 0)),
          pl.BlockSpec((_DH, 128), lambda i: (0, 0)),
          pl.BlockSpec((1, 1), lambda i: (0, 0)),
      ],
      out_specs=pl.BlockSpec((m, 128), lambda i: (i, 0)),
      out_shape=jax.ShapeDtypeStruct((_N, 128), jnp.float32),
      name="tc_main2",
  )(a2, cnt, zr, w2n, wc, bc)


def kernel(x, edge_index, W1_neigh, W1_root, b1, W2_neigh, W2_root, b2, Wc, bc):
  src = edge_index[0]
  dst = edge_index[1]
  e = src.shape[0]
  n_chunks = -(-e // (_NS * _B * 2 * _SG)) * 2 * _SG
  pad = n_chunks * _NS * _B - e
  if pad:
    src = jnp.concatenate([src, jnp.zeros((pad,), jnp.int32)])
    dst = jnp.concatenate([dst, jnp.full((pad,), _N, jnp.int32)])
  src3 = src.reshape(_NS, n_chunks, _B)
  dst3 = dst.reshape(_NS, n_chunks, _B)

  def edge_rows(f_blocks):
    off = (jnp.arange(f_blocks, dtype=jnp.int32) * _N)[:, None, None, None]
    sa = src3[None] + off
    da = jnp.broadcast_to(dst3[None], sa.shape)
    return jnp.stack([sa, da], axis=3).reshape(-1, 2, _B)

  ei1 = edge_rows(2)
  ei2 = edge_rows(4)
  zf = jnp.zeros((_RPT, 128), jnp.float32)
  ones = jnp.ones((_B, 128), jnp.float32)
  xb = x.reshape(_N, 2, 128).transpose(1, 0, 2).reshape(2 * _N, 128)

  out1 = _sc_segment_sum(2, n_chunks, True)(xb, ei1, zf, ones)
  zr1 = _tc_root(x, W1_root, b1.reshape(1, _DH), _DIN)
  h1b = _tc_main1(out1, out1, zr1, W1_neigh)
  h1flat = h1b.reshape(4 * _N, 128)
  agg2 = _sc_segment_sum(4, n_chunks, False)(h1flat, ei2, zf, ones)
  zr2 = _tc_root2(h1b, W2_root, b2.reshape(1, _DH))
  wc_pad = jnp.pad(Wc, ((0, 0), (0, 127)))
  out = _tc_main2(agg2, out1, zr2, W2_neigh, wc_pad, bc.reshape(1, 1))
  return out[:, 0]


# speedup vs baseline: 1.0916x; 1.0179x over previous
import jax
import jax.numpy as jnp
from jax import lax
from jax.experimental import pallas as pl
from jax.experimental.pallas import tpu as pltpu
from jax.experimental.pallas import tpu_sc as plsc

_N = 10000
_DIN = 256
_DH = 512
_NC = 2
_NS = 16
_B = 128
_NSLOT = 2
_SG = 4
_NACC = 10112
_RPT = _NACC // _NS
_MBLK = 1000


def _sc_segment_sum(n_blocks, n_chunks, with_cnt):
  feat_rounds = n_blocks // _NC
  rounds = feat_rounds + (1 if with_cnt else 0)
  n_sg = n_chunks // _SG
  span = 2 * _SG

  def body(h_ref, ei_ref, zf_ref, ones_ref, agg_out,
           ibuf, msg, *sems_and_agg):
    gsem = sems_and_agg[:_NSLOT]
    ssem = sems_and_agg[_NSLOT:2 * _NSLOT]
    agg_sh = sems_and_agg[-1]
    cid = lax.axis_index("c")
    sid = lax.axis_index("s")
    r0 = sid * _RPT

    def run_gather_block(f):
      ebase = (f * _NS + sid) * n_chunks

      def it(t, carry):
        for q in range(2):
          pltpu.sync_copy(
              ei_ref.at[pl.ds(ebase + (2 * t + q) * _SG, _SG)], ibuf.at[q])
        gd = {}
        sd = {}
        for k in range(_NSLOT):
          q, j = divmod(k, _SG)
          gd[k] = pltpu.async_copy(
              h_ref.at[ibuf.at[q, j, 0]], msg.at[k], gsem[k])
        for k in range(span):
          q, j = divmod(k, _SG)
          s = k % _NSLOT
          gd[k].wait()
          sd[k] = pltpu.async_copy(
              msg.at[s], agg_sh.at[ibuf.at[q, j, 1]], ssem[s], add=True)
          if k + _NSLOT < span:
            sd[k].wait()
            nq, nj = divmod(k + _NSLOT, _SG)
            gd[k + _NSLOT] = pltpu.async_copy(
                h_ref.at[ibuf.at[nq, nj, 0]], msg.at[s], gsem[s])
        for k in range(span - _NSLOT, span):
          sd[k].wait()
        return carry
      lax.fori_loop(0, n_sg // 2, it, 0)

    def run_cnt(c_lo, c_hi):
      ebase = sid * n_chunks
      pltpu.sync_copy(ones_ref, msg.at[0])

      def it(t, carry):
        pltpu.sync_copy(
            ei_ref.at[pl.ds(ebase + c_lo + t * _SG, _SG)], ibuf.at[0])
        descs = [pltpu.async_copy(msg.at[0], agg_sh.at[ibuf.at[0, j, 1]],
                                  ssem[0], add=True) for j in range(_SG)]
        for d in descs:
          d.wait()
        return carry
      lax.fori_loop(0, (c_hi - c_lo) // _SG, it, 0)

    for r in range(rounds):
      pltpu.sync_copy(zf_ref, agg_sh.at[pl.ds(r0, _RPT)])
      plsc.subcore_barrier()
      is_cnt = with_cnt and r == feat_rounds
      if is_cnt:
        half = n_chunks // 2
        pl.when(cid == 0)(lambda: run_cnt(0, half))
        pl.when(cid == 1)(lambda: run_cnt(half, n_chunks))
      else:
        pl.when(cid == 0)(lambda: run_gather_block(2 * r))
        pl.when(cid == 1)(lambda: run_gather_block(2 * r + 1))
      plsc.subcore_barrier()
      for c in range(_NC):
        f = 2 * r + c
        pl.when(cid == c)(lambda f=f: pltpu.sync_copy(
            agg_sh.at[pl.ds(r0, _RPT)], agg_out.at[f].at[pl.ds(r0, _RPT)]))

  out_type = jax.ShapeDtypeStruct((2 * rounds, _NACC, 128), jnp.float32)
  scratch = (
      [pltpu.VMEM((2, _SG, 2, _B), jnp.int32),
       pltpu.VMEM((_NSLOT, _B, 128), jnp.float32)]
      + [pltpu.SemaphoreType.DMA] * (2 * _NSLOT)
      + [pltpu.VMEM_SHARED((_NACC, 128), jnp.float32)]
  )
  mesh = plsc.VectorSubcoreMesh(core_axis_name="c", subcore_axis_name="s")
  return pl.kernel(body, out_type=out_type, mesh=mesh,
                   scratch_types=scratch, name=f"sc_segsum_{n_blocks}")


def _tc_layer1(a1, cnt, x, w1n, w1r, b1):
  m = _MBLK

  def body(a_ref, c_ref, x_ref, wn_ref, wr_ref, b_ref, out_ref):
    cnt = c_ref[0][:, :1] + c_ref[1][:, :1]
    inv = 1.0 / jnp.maximum(cnt, 1.0)
    agg = jnp.concatenate([a_ref[0], a_ref[1]], axis=1) * inv
    z = (jnp.dot(agg, wn_ref[...], preferred_element_type=jnp.float32)
         + jnp.dot(x_ref[...], wr_ref[...], preferred_element_type=jnp.float32)
         + b_ref[...])
    h = jnp.maximum(z, 0.0)
    for j in range(4):
      out_ref[j] = h[:, j * 128:(j + 1) * 128]

  return pl.pallas_call(
      body,
      grid=(_N // m,),
      in_specs=[
          pl.BlockSpec((2, m, 128), lambda i: (0, i, 0)),
          pl.BlockSpec((2, m, 128), lambda i: (1, i, 0)),
          pl.BlockSpec((m, _DIN), lambda i: (i, 0)),
          pl.BlockSpec((_DIN, _DH), lambda i: (0, 0)),
          pl.BlockSpec((_DIN, _DH), lambda i: (0, 0)),
          pl.BlockSpec((1, _DH), lambda i: (0, 0)),
      ],
      out_specs=pl.BlockSpec((4, m, 128), lambda i: (0, i, 0)),
      out_shape=jax.ShapeDtypeStruct((4, _N, 128), jnp.float32),
      name="tc_layer1",
  )(a1, cnt, x, w1n, w1r, b1)


def _tc_layer2(a2, cnt, h1b, w2n, w2r, b2, wc, bc):
  m = _MBLK

  def body(a_ref, c_ref, h_ref, wn_ref, wr_ref, b_ref, wc_ref, bc_ref,
           out_ref):
    cnt = c_ref[0][:, :1] + c_ref[1][:, :1]
    inv = 1.0 / jnp.maximum(cnt, 1.0)
    agg = jnp.concatenate([a_ref[j] for j in range(4)], axis=1) * inv
    h1 = jnp.concatenate([h_ref[j] for j in range(4)], axis=1)
    z = (jnp.dot(agg, wn_ref[...], preferred_element_type=jnp.float32)
         + jnp.dot(h1, wr_ref[...], preferred_element_type=jnp.float32)
         + b_ref[...])
    h2 = jnp.maximum(z, 0.0)
    out_ref[...] = (jnp.dot(h2, wc_ref[...], preferred_element_type=jnp.float32)
                    + bc_ref[0, 0])

  return pl.pallas_call(
      body,
      grid=(_N // m,),
      in_specs=[
          pl.BlockSpec((4, m, 128), lambda i: (0, i, 0)),
          pl.BlockSpec((2, m, 128), lambda i: (1, i, 0)),
          pl.BlockSpec((4, m, 128), lambda i: (0, i, 0)),
          pl.BlockSpec((_DH, _DH), lambda i: (0, 0)),
          pl.BlockSpec((_DH, _DH), lambda i: (0, 0)),
          pl.BlockSpec((1, _DH), lambda i: (0, 0)),
          pl.BlockSpec((_DH, 128), lambda i: (0, 0)),
          pl.BlockSpec((1, 1), lambda i: (0, 0)),
      ],
      out_specs=pl.BlockSpec((m, 128), lambda i: (i, 0)),
      out_shape=jax.ShapeDtypeStruct((_N, 128), jnp.float32),
      name="tc_layer2",
  )(a2, cnt, h1b, w2n, w2r, b2, wc, bc)


def kernel(x, edge_index, W1_neigh, W1_root, b1, W2_neigh, W2_root, b2, Wc, bc):
  src = edge_index[0]
  dst = edge_index[1]
  e = src.shape[0]
  n_chunks = -(-e // (_NS * _B * 2 * _SG)) * 2 * _SG
  pad = n_chunks * _NS * _B - e
  if pad:
    src = jnp.concatenate([src, jnp.zeros((pad,), jnp.int32)])
    dst = jnp.concatenate([dst, jnp.full((pad,), _N, jnp.int32)])
  src3 = src.reshape(_NS, n_chunks, _B)
  dst3 = dst.reshape(_NS, n_chunks, _B)

  def edge_rows(f_blocks):
    off = (jnp.arange(f_blocks, dtype=jnp.int32) * _N)[:, None, None, None]
    sa = src3[None] + off
    da = jnp.broadcast_to(dst3[None], sa.shape)
    return jnp.stack([sa, da], axis=3).reshape(-1, 2, _B)

  ei1 = edge_rows(2)
  ei2 = edge_rows(4)
  zf = jnp.zeros((_RPT, 128), jnp.float32)
  ones = jnp.ones((_B, 128), jnp.float32)
  xb = x.reshape(_N, 2, 128).transpose(1, 0, 2).reshape(2 * _N, 128)

  out1 = _sc_segment_sum(2, n_chunks, True)(xb, ei1, zf, ones)
  h1b = _tc_layer1(out1, out1, x, W1_neigh, W1_root, b1.reshape(1, _DH))
  agg2 = _sc_segment_sum(4, n_chunks, False)(
      h1b.reshape(4 * _N, 128), ei2, zf, ones)
  wc_pad = jnp.pad(Wc, ((0, 0), (0, 127)))
  out = _tc_layer2(agg2, out1, h1b, W2_neigh, W2_root, b2.reshape(1, _DH),
                   wc_pad, bc.reshape(1, 1))
  return out[:, 0]


# speedup vs baseline: 1.1292x; 1.0345x over previous
import jax
import jax.numpy as jnp
from jax import lax
from jax.experimental import pallas as pl
from jax.experimental.pallas import tpu as pltpu
from jax.experimental.pallas import tpu_sc as plsc

_N = 10000
_DIN = 256
_DH = 512
_NC = 2
_NS = 16
_B = 128
_NSLOT = 2
_SG = 8
_NACC = 10112
_RPT = _NACC // _NS
_MBLK = 1000


def _sc_segment_sum(n_blocks, n_chunks, with_cnt):
  feat_rounds = n_blocks // _NC
  rounds = feat_rounds + (1 if with_cnt else 0)
  n_sg = n_chunks // _SG
  span = 2 * _SG

  def body(h_ref, ei_ref, zf_ref, ones_ref, agg_out,
           ibuf, msg, *sems_and_agg):
    gsem = sems_and_agg[:_NSLOT]
    ssem = sems_and_agg[_NSLOT:2 * _NSLOT]
    agg_sh = sems_and_agg[-1]
    cid = lax.axis_index("c")
    sid = lax.axis_index("s")
    r0 = sid * _RPT

    def run_gather_block(f):
      ebase = (f * _NS + sid) * n_chunks

      def it(t, carry):
        for q in range(2):
          pltpu.sync_copy(
              ei_ref.at[pl.ds(ebase + (2 * t + q) * _SG, _SG)], ibuf.at[q])
        gd = {}
        sd = {}
        for k in range(_NSLOT):
          q, j = divmod(k, _SG)
          gd[k] = pltpu.async_copy(
              h_ref.at[ibuf.at[q, j, 0]], msg.at[k], gsem[k])
        for k in range(span):
          q, j = divmod(k, _SG)
          s = k % _NSLOT
          gd[k].wait()
          sd[k] = pltpu.async_copy(
              msg.at[s], agg_sh.at[ibuf.at[q, j, 1]], ssem[s], add=True)
          if k + _NSLOT < span:
            sd[k].wait()
            nq, nj = divmod(k + _NSLOT, _SG)
            gd[k + _NSLOT] = pltpu.async_copy(
                h_ref.at[ibuf.at[nq, nj, 0]], msg.at[s], gsem[s])
        for k in range(span - _NSLOT, span):
          sd[k].wait()
        return carry
      lax.fori_loop(0, n_sg // 2, it, 0)

    def run_cnt(c_lo, c_hi):
      ebase = sid * n_chunks
      pltpu.sync_copy(ones_ref, msg.at[0])

      def it(t, carry):
        pltpu.sync_copy(
            ei_ref.at[pl.ds(ebase + c_lo + t * _SG, _SG)], ibuf.at[0])
        descs = [pltpu.async_copy(msg.at[0], agg_sh.at[ibuf.at[0, j, 1]],
                                  ssem[0], add=True) for j in range(_SG)]
        for d in descs:
          d.wait()
        return carry
      lax.fori_loop(0, (c_hi - c_lo) // _SG, it, 0)

    for r in range(rounds):
      pltpu.sync_copy(zf_ref, agg_sh.at[pl.ds(r0, _RPT)])
      plsc.subcore_barrier()
      is_cnt = with_cnt and r == feat_rounds
      if is_cnt:
        half = n_chunks // 2
        pl.when(cid == 0)(lambda: run_cnt(0, half))
        pl.when(cid == 1)(lambda: run_cnt(half, n_chunks))
      else:
        pl.when(cid == 0)(lambda: run_gather_block(2 * r))
        pl.when(cid == 1)(lambda: run_gather_block(2 * r + 1))
      plsc.subcore_barrier()
      for c in range(_NC):
        f = 2 * r + c
        pl.when(cid == c)(lambda f=f: pltpu.sync_copy(
            agg_sh.at[pl.ds(r0, _RPT)], agg_out.at[f].at[pl.ds(r0, _RPT)]))

  out_type = jax.ShapeDtypeStruct((2 * rounds, _NACC, 128), jnp.float32)
  scratch = (
      [pltpu.VMEM((2, _SG, 2, _B), jnp.int32),
       pltpu.VMEM((_NSLOT, _B, 128), jnp.float32)]
      + [pltpu.SemaphoreType.DMA] * (2 * _NSLOT)
      + [pltpu.VMEM_SHARED((_NACC, 128), jnp.float32)]
  )
  mesh = plsc.VectorSubcoreMesh(core_axis_name="c", subcore_axis_name="s")
  return pl.kernel(body, out_type=out_type, mesh=mesh,
                   scratch_types=scratch, name=f"sc_segsum_{n_blocks}")


def _tc_layer1(a1, cnt, x, w1n, w1r, b1):
  m = _MBLK

  def body(a_ref, c_ref, x_ref, wn_ref, wr_ref, b_ref, out_ref):
    cnt = c_ref[0][:, :1] + c_ref[1][:, :1]
    inv = 1.0 / jnp.maximum(cnt, 1.0)
    agg = jnp.concatenate([a_ref[0], a_ref[1]], axis=1) * inv
    z = (jnp.dot(agg, wn_ref[...], preferred_element_type=jnp.float32)
         + jnp.dot(x_ref[...], wr_ref[...], preferred_element_type=jnp.float32)
         + b_ref[...])
    h = jnp.maximum(z, 0.0)
    for j in range(4):
      out_ref[j] = h[:, j * 128:(j + 1) * 128]

  return pl.pallas_call(
      body,
      grid=(_N // m,),
      in_specs=[
          pl.BlockSpec((2, m, 128), lambda i: (0, i, 0)),
          pl.BlockSpec((2, m, 128), lambda i: (1, i, 0)),
          pl.BlockSpec((m, _DIN), lambda i: (i, 0)),
          pl.BlockSpec((_DIN, _DH), lambda i: (0, 0)),
          pl.BlockSpec((_DIN, _DH), lambda i: (0, 0)),
          pl.BlockSpec((1, _DH), lambda i: (0, 0)),
      ],
      out_specs=pl.BlockSpec((4, m, 128), lambda i: (0, i, 0)),
      out_shape=jax.ShapeDtypeStruct((4, _N, 128), jnp.float32),
      name="tc_layer1",
  )(a1, cnt, x, w1n, w1r, b1)


def _tc_layer2(a2, cnt, h1b, w2n, w2r, b2, wc, bc):
  m = _MBLK

  def body(a_ref, c_ref, h_ref, wn_ref, wr_ref, b_ref, wc_ref, bc_ref,
           out_ref):
    cnt = c_ref[0][:, :1] + c_ref[1][:, :1]
    inv = 1.0 / jnp.maximum(cnt, 1.0)
    agg = jnp.concatenate([a_ref[j] for j in range(4)], axis=1) * inv
    h1 = jnp.concatenate([h_ref[j] for j in range(4)], axis=1)
    z = (jnp.dot(agg, wn_ref[...], preferred_element_type=jnp.float32)
         + jnp.dot(h1, wr_ref[...], preferred_element_type=jnp.float32)
         + b_ref[...])
    h2 = jnp.maximum(z, 0.0)
    out_ref[...] = (jnp.dot(h2, wc_ref[...], preferred_element_type=jnp.float32)
                    + bc_ref[0, 0])

  return pl.pallas_call(
      body,
      grid=(_N // m,),
      in_specs=[
          pl.BlockSpec((4, m, 128), lambda i: (0, i, 0)),
          pl.BlockSpec((2, m, 128), lambda i: (1, i, 0)),
          pl.BlockSpec((4, m, 128), lambda i: (0, i, 0)),
          pl.BlockSpec((_DH, _DH), lambda i: (0, 0)),
          pl.BlockSpec((_DH, _DH), lambda i: (0, 0)),
          pl.BlockSpec((1, _DH), lambda i: (0, 0)),
          pl.BlockSpec((_DH, 128), lambda i: (0, 0)),
          pl.BlockSpec((1, 1), lambda i: (0, 0)),
      ],
      out_specs=pl.BlockSpec((m, 128), lambda i: (i, 0)),
      out_shape=jax.ShapeDtypeStruct((_N, 128), jnp.float32),
      name="tc_layer2",
  )(a2, cnt, h1b, w2n, w2r, b2, wc, bc)


def kernel(x, edge_index, W1_neigh, W1_root, b1, W2_neigh, W2_root, b2, Wc, bc):
  src = edge_index[0]
  dst = edge_index[1]
  e = src.shape[0]
  n_chunks = -(-e // (_NS * _B * 2 * _SG)) * 2 * _SG
  pad = n_chunks * _NS * _B - e
  if pad:
    src = jnp.concatenate([src, jnp.zeros((pad,), jnp.int32)])
    dst = jnp.concatenate([dst, jnp.full((pad,), _N, jnp.int32)])
  src3 = src.reshape(_NS, n_chunks, _B)
  dst3 = dst.reshape(_NS, n_chunks, _B)

  def edge_rows(f_blocks):
    off = (jnp.arange(f_blocks, dtype=jnp.int32) * _N)[:, None, None, None]
    sa = src3[None] + off
    da = jnp.broadcast_to(dst3[None], sa.shape)
    return jnp.stack([sa, da], axis=3).reshape(-1, 2, _B)

  ei1 = edge_rows(2)
  ei2 = edge_rows(4)
  zf = jnp.zeros((_RPT, 128), jnp.float32)
  ones = jnp.ones((_B, 128), jnp.float32)
  xb = x.reshape(_N, 2, 128).transpose(1, 0, 2).reshape(2 * _N, 128)

  out1 = _sc_segment_sum(2, n_chunks, True)(xb, ei1, zf, ones)
  h1b = _tc_layer1(out1, out1, x, W1_neigh, W1_root, b1.reshape(1, _DH))
  agg2 = _sc_segment_sum(4, n_chunks, False)(
      h1b.reshape(4 * _N, 128), ei2, zf, ones)
  wc_pad = jnp.pad(Wc, ((0, 0), (0, 127)))
  out = _tc_layer2(agg2, out1, h1b, W2_neigh, W2_root, b2.reshape(1, _DH),
                   wc_pad, bc.reshape(1, 1))
  return out[:, 0]
